# Initial kernel scaffold; baseline (speedup 1.0000x reference)
import jax
import jax.numpy as jnp
from jax.experimental import pallas as pl


def kernel(x, table, W, b):
    raise NotImplementedError("write your pallas kernel here")


# speedup vs baseline: 1.4705x; 1.4705x over previous
import functools

import jax
import jax.numpy as jnp
from jax import lax
from jax.experimental import pallas as pl
from jax.experimental.pallas import tpu as pltpu
from jax.experimental.pallas import tpu_sc as plsc

_NC, _NS, _L = 2, 16, 16
_NW = _NC * _NS
_B, _H = 16384, 50
_V, _D = 1000000, 16
_EPW = _B // _NW
_IPW = _EPW * _H
_ROWS = _V * _D // 128
_TC_BLK = 5000


def _tw_body(t_ref, w_ref, b_ref, o_ref):
    o_ref[...] = (
        jnp.dot(t_ref[...], w_ref[...], preferred_element_type=jnp.float32)
        + b_ref[0]
    )


def _tw_call(tr, wm, b):
    return pl.pallas_call(
        _tw_body,
        grid=(_ROWS // _TC_BLK,),
        in_specs=[
            pl.BlockSpec((_TC_BLK, 128), lambda i: (i, 0)),
            pl.BlockSpec((128, 8), lambda i: (0, 0)),
            pl.BlockSpec(memory_space=pltpu.SMEM),
        ],
        out_specs=pl.BlockSpec((_TC_BLK, 8), lambda i: (i, 0)),
        out_shape=jax.ShapeDtypeStruct((_ROWS, 8), jnp.float32),
    )(tr, wm, b)


_mesh = plsc.VectorSubcoreMesh(core_axis_name="c", subcore_axis_name="s")


@functools.partial(
    pl.kernel,
    out_type=jax.ShapeDtypeStruct((_NW, _L), jnp.float32),
    mesh=_mesh,
    scratch_types=[
        pltpu.VMEM((_IPW,), jnp.int32),
        pltpu.VMEM((_IPW,), jnp.float32),
        pltpu.VMEM((_L,), jnp.float32),
        pltpu.SemaphoreType.DMA,
    ],
)
def _sc_body(xf_hbm, tw_hbm, out_hbm, idx_v, g_v, o_v, sem):
    wid = lax.axis_index("s") * _NC + lax.axis_index("c")
    pltpu.sync_copy(xf_hbm.at[pl.ds(wid * _IPW, _IPW)], idx_v)
    pltpu.async_copy(tw_hbm.at[idx_v], g_v, sem).wait()

    def outer(c, tot):
        base = c * _L

        def inner(j, acc):
            return acc + g_v[pl.ds(j * _EPW + base, _L)]

        s = lax.fori_loop(0, _H, inner, jnp.zeros((_L,), jnp.float32))
        m = s * (1.0 / _H)
        return tot + m * m

    tot = lax.fori_loop(0, _EPW // _L, outer, jnp.zeros((_L,), jnp.float32))
    o_v[...] = tot
    pltpu.sync_copy(o_v, out_hbm.at[wid])


def kernel(x, table, W, b):
    tr = table.reshape(_ROWS, 128)
    wm = jnp.kron(jnp.eye(8, dtype=jnp.float32), W)
    tw = _tw_call(tr, wm, b).reshape(-1)
    xf = (
        x.reshape(_NW, _EPW, _H)
        .transpose(0, 2, 1)
        .reshape(-1)
        .astype(jnp.int32)
    )
    parts = _sc_body(xf, tw)
    return jnp.sum(parts) * 128.0
